# baseline (speedup 1.0000x reference)
import jax
import jax.numpy as jnp
from jax import lax
from jax.experimental import pallas as pl
from jax.experimental.pallas import tpu as pltpu
from jax.experimental.pallas import tpu_sc as plsc

NGRAMS = 20
BATCH = 1024
EMB_DIM = 64
LANES = 16
NW = 32
BPW = BATCH // NW
IDX_PER_W = BPW * NGRAMS
GCHUNK = 128
NCHUNK = IDX_PER_W // GCHUNK
ZROW = IDX_PER_W
NGROUP = BPW // LANES
NDC = EMB_DIM // LANES
TBL_ROWS = 51200

_TAKE_DNUMS = lax.GatherDimensionNumbers(
    offset_dims=(), collapsed_slice_dims=(0,), start_index_map=(0,)
)


def _take(vec, idx):
    return lax.gather(
        vec,
        idx[:, None],
        _TAKE_DNUMS,
        (1,),
        mode=lax.GatherScatterMode.PROMISE_IN_BOUNDS,
    )


def _sc_body(idx_hbm, table_hbm, out_hbm, idx_v, gidx_v, rows_v, out_v, sem,
             gsem):
    wid = lax.axis_index("s") * 2 + lax.axis_index("c")
    base = wid * BPW

    idx_copies = [
        pltpu.make_async_copy(
            idx_hbm.at[pl.ds(s * BATCH + base, BPW)],
            idx_v.at[pl.ds(s * BPW, BPW)],
            sem,
        )
        for s in range(NGRAMS)
    ]
    with jax.named_scope("stage_idx"):
        for c in idx_copies:
            c.start()
        for c in idx_copies:
            c.wait()

    lanes = lax.iota(jnp.int32, LANES)
    with jax.named_scope("gidx"):
        for s in range(NGRAMS):
            for g in range(NGROUP):
                x = idx_v[pl.ds(s * BPW + g * LANES, LANES)]
                plsc.store_scatter(
                    gidx_v,
                    [lanes * NGRAMS + (g * LANES * NGRAMS + s)],
                    jnp.where(x >= TBL_ROWS, x - TBL_ROWS, x),
                )

    row_copies = [
        pltpu.make_async_copy(
            table_hbm.at[gidx_v.at[pl.ds(j * GCHUNK, GCHUNK)]],
            rows_v.at[pl.ds(j * GCHUNK, GCHUNK)],
            gsem,
        )
        for j in range(NCHUNK)
    ]
    for c in row_copies:
        c.start()

    zeros16 = jnp.zeros((LANES,), jnp.float32)
    for dc in range(2 * EMB_DIM // LANES):
        rows_v[ZROW, pl.ds(dc * LANES, LANES)] = zeros16

    with jax.named_scope("masks"):
        groups = []
        for g in range(NGROUP):
            v = [
                idx_v[pl.ds(s * BPW + g * LANES, LANES)] for s in range(NGRAMS)
            ]
            pf = []
            cb = []
            for s in range(NGRAMS):
                dup = None
                for t in range(s):
                    e = v[s] == v[t]
                    dup = e if dup is None else (dup | e)
                row = lanes * NGRAMS + (g * LANES * NGRAMS + s)
                if dup is not None:
                    row = jnp.where(dup, ZROW, row)
                pf.append(row)
                cb.append(jnp.where(v[s] >= TBL_ROWS, EMB_DIM, 0))
            groups.append((pf, cb))

    coffs = [dc * LANES + lanes for dc in range(NDC)]
    waited = 0
    for g in range(NGROUP):
        need = min(NCHUNK, -(-((g + 1) * LANES * NGRAMS) // GCHUNK))
        with jax.named_scope("gather_wait"):
            while waited < need:
                row_copies[waited].wait()
                waited += 1
        pf, cb = groups[g]
        with jax.named_scope("accum"):

            def bbody(b, _, pf=pf, cb=cb, g=g):
                bsp = jnp.full((LANES,), b, jnp.int32)
                accs = None
                for s in range(NGRAMS):
                    rsp = _take(pf[s], bsp)
                    csp = _take(cb[s], bsp)
                    vals = [
                        plsc.load_gather(rows_v, [rsp, csp + coff])
                        for coff in coffs
                    ]
                    if accs is None:
                        accs = vals
                    else:
                        accs = [a + x for a, x in zip(accs, vals)]
                for dc in range(NDC):
                    out_v[b + g * LANES, pl.ds(dc * LANES, LANES)] = accs[dc]
                return _

            lax.fori_loop(0, LANES, bbody, None, unroll=2)

    with jax.named_scope("writeout"):
        pltpu.sync_copy(out_v, out_hbm.at[pl.ds(base, BPW)])


def _pack_body(a_ref, b_ref, out_ref):
    ab = jnp.concatenate([a_ref[...], b_ref[...]], axis=0)
    eye = jnp.eye(2 * EMB_DIM, dtype=jnp.float32)
    dn = (((0,), (0,)), ((), ()))
    out_ref[...] = lax.dot_general(
        ab, eye, dn, preferred_element_type=jnp.float32
    )


_PACK_CB = 12800


def _pack(Wt):
    nblk = TBL_ROWS // _PACK_CB
    return pl.pallas_call(
        _pack_body,
        grid=(nblk,),
        in_specs=[
            pl.BlockSpec((EMB_DIM, _PACK_CB), lambda i: (0, i)),
            pl.BlockSpec((EMB_DIM, _PACK_CB), lambda i, n=nblk: (0, i + n)),
        ],
        out_specs=pl.BlockSpec((_PACK_CB, 2 * EMB_DIM), lambda i: (i, 0)),
        out_shape=jax.ShapeDtypeStruct((TBL_ROWS, 2 * EMB_DIM), jnp.float32),
        compiler_params=pltpu.CompilerParams(
            dimension_semantics=("parallel",)
        ),
    )(Wt, Wt)


def kernel(input, W):
    idx_lin = jnp.maximum(input.reshape(-1), 0)
    table2 = _pack(W.T)
    mesh = plsc.VectorSubcoreMesh(core_axis_name="c", subcore_axis_name="s")
    f = pl.kernel(
        _sc_body,
        out_type=jax.ShapeDtypeStruct((BATCH, EMB_DIM), jnp.float32),
        mesh=mesh,
        compiler_params=pltpu.CompilerParams(
            needs_layout_passes=False, use_tc_tiling_on_sc=False
        ),
        scratch_types=[
            pltpu.VMEM((IDX_PER_W,), jnp.int32),
            pltpu.VMEM((IDX_PER_W,), jnp.int32),
            pltpu.VMEM((IDX_PER_W + 1, 2 * EMB_DIM), jnp.float32),
            pltpu.VMEM((BPW, EMB_DIM), jnp.float32),
            pltpu.SemaphoreType.DMA,
            pltpu.SemaphoreType.DMA,
        ],
    )
    return f(idx_lin, table2)


# speedup vs baseline: 1.0111x; 1.0111x over previous
import jax
import jax.numpy as jnp
from jax import lax
from jax.experimental import pallas as pl
from jax.experimental.pallas import tpu as pltpu
from jax.experimental.pallas import tpu_sc as plsc

NGRAMS = 20
BATCH = 1024
EMB_DIM = 64
LANES = 16
NW = 32
BPW = BATCH // NW
IDX_PER_W = BPW * NGRAMS
GCHUNK = 128
NCHUNK = IDX_PER_W // GCHUNK
ZROW = IDX_PER_W
NGROUP = BPW // LANES
NDC = EMB_DIM // LANES
TBL_ROWS = 51200

_TAKE_DNUMS = lax.GatherDimensionNumbers(
    offset_dims=(), collapsed_slice_dims=(0,), start_index_map=(0,)
)


def _take(vec, idx):
    return lax.gather(
        vec,
        idx[:, None],
        _TAKE_DNUMS,
        (1,),
        mode=lax.GatherScatterMode.PROMISE_IN_BOUNDS,
    )


def _sc_body(idx_hbm, table_hbm, out_hbm, idx_v, gidx_v, rows_v, out_v, sem,
             gsem):
    wid = lax.axis_index("s") * 2 + lax.axis_index("c")
    base = wid * BPW

    idx_copies = [
        pltpu.make_async_copy(
            idx_hbm.at[pl.ds(s * BATCH + base, BPW)],
            idx_v.at[pl.ds(s * BPW, BPW)],
            sem,
        )
        for s in range(NGRAMS)
    ]
    with jax.named_scope("stage_idx"):
        for c in idx_copies:
            c.start()
        for c in idx_copies:
            c.wait()

    lanes = lax.iota(jnp.int32, LANES)
    with jax.named_scope("gidx"):
        for s in range(NGRAMS):
            for g in range(NGROUP):
                x = idx_v[pl.ds(s * BPW + g * LANES, LANES)]
                plsc.store_scatter(
                    gidx_v,
                    [lanes * NGRAMS + (g * LANES * NGRAMS + s)],
                    jnp.where(x >= TBL_ROWS, x - TBL_ROWS, x),
                )

    row_copies = [
        pltpu.make_async_copy(
            table_hbm.at[gidx_v.at[pl.ds(j * GCHUNK, GCHUNK)]],
            rows_v.at[pl.ds(j * GCHUNK, GCHUNK)],
            gsem,
        )
        for j in range(NCHUNK)
    ]
    for c in row_copies:
        c.start()

    zeros16 = jnp.zeros((LANES,), jnp.float32)
    for dc in range(2 * EMB_DIM // LANES):
        rows_v[ZROW, pl.ds(dc * LANES, LANES)] = zeros16

    with jax.named_scope("masks"):
        groups = []
        for g in range(NGROUP):
            v = [
                idx_v[pl.ds(s * BPW + g * LANES, LANES)] for s in range(NGRAMS)
            ]
            pf = []
            cb = []
            for s in range(NGRAMS):
                dup = None
                for t in range(s):
                    e = v[s] == v[t]
                    dup = e if dup is None else (dup | e)
                row = lanes * NGRAMS + (g * LANES * NGRAMS + s)
                if dup is not None:
                    row = jnp.where(dup, ZROW, row)
                pf.append(row)
                cb.append(jnp.where(v[s] >= TBL_ROWS, EMB_DIM, 0))
            groups.append((pf, cb))

    coffs = [dc * LANES + lanes for dc in range(NDC)]
    waited = 0
    for g in range(NGROUP):
        need = min(NCHUNK, -(-((g + 1) * LANES * NGRAMS) // GCHUNK))
        with jax.named_scope("gather_wait"):
            while waited < need:
                row_copies[waited].wait()
                waited += 1
        pf, cb = groups[g]
        with jax.named_scope("accum"):

            def bbody(b, _, pf=pf, cb=cb, g=g):
                bsp = jnp.full((LANES,), b, jnp.int32)
                accs = None
                for s in range(NGRAMS):
                    rsp = _take(pf[s], bsp)
                    csp = _take(cb[s], bsp)
                    vals = [
                        plsc.load_gather(rows_v, [rsp, csp + coff])
                        for coff in coffs
                    ]
                    if accs is None:
                        accs = vals
                    else:
                        accs = [a + x for a, x in zip(accs, vals)]
                for dc in range(NDC):
                    out_v[b + g * LANES, pl.ds(dc * LANES, LANES)] = accs[dc]
                return _

            lax.fori_loop(0, LANES, bbody, None)

    with jax.named_scope("writeout"):
        pltpu.sync_copy(out_v, out_hbm.at[pl.ds(base, BPW)])


def _pack_body(a_ref, b_ref, out_ref):
    ab = jnp.concatenate([a_ref[...], b_ref[...]], axis=0)
    eye = jnp.eye(2 * EMB_DIM, dtype=jnp.float32)
    dn = (((0,), (0,)), ((), ()))
    out_ref[...] = lax.dot_general(
        ab, eye, dn, preferred_element_type=jnp.float32
    )


_PACK_CB = 10240


def _pack(Wt):
    nblk = TBL_ROWS // _PACK_CB
    return pl.pallas_call(
        _pack_body,
        grid=(nblk,),
        in_specs=[
            pl.BlockSpec((EMB_DIM, _PACK_CB), lambda i: (0, i)),
            pl.BlockSpec((EMB_DIM, _PACK_CB), lambda i, n=nblk: (0, i + n)),
        ],
        out_specs=pl.BlockSpec((_PACK_CB, 2 * EMB_DIM), lambda i: (i, 0)),
        out_shape=jax.ShapeDtypeStruct((TBL_ROWS, 2 * EMB_DIM), jnp.float32),
        compiler_params=pltpu.CompilerParams(
            dimension_semantics=("parallel",)
        ),
    )(Wt, Wt)


def kernel(input, W):
    idx_lin = jnp.maximum(input.reshape(-1), 0)
    table2 = _pack(W.T)
    mesh = plsc.VectorSubcoreMesh(core_axis_name="c", subcore_axis_name="s")
    f = pl.kernel(
        _sc_body,
        out_type=jax.ShapeDtypeStruct((BATCH, EMB_DIM), jnp.float32),
        mesh=mesh,
        compiler_params=pltpu.CompilerParams(
            needs_layout_passes=False, use_tc_tiling_on_sc=False
        ),
        scratch_types=[
            pltpu.VMEM((IDX_PER_W,), jnp.int32),
            pltpu.VMEM((IDX_PER_W,), jnp.int32),
            pltpu.VMEM((IDX_PER_W + 1, 2 * EMB_DIM), jnp.float32),
            pltpu.VMEM((BPW, EMB_DIM), jnp.float32),
            pltpu.SemaphoreType.DMA,
            pltpu.SemaphoreType.DMA,
        ],
    )
    return f(idx_lin, table2)


# speedup vs baseline: 1.0136x; 1.0025x over previous
import jax
import jax.numpy as jnp
from jax import lax
from jax.experimental import pallas as pl
from jax.experimental.pallas import tpu as pltpu
from jax.experimental.pallas import tpu_sc as plsc

NGRAMS = 20
BATCH = 1024
EMB_DIM = 64
LANES = 16
NW = 32
BPW = BATCH // NW
IDX_PER_W = BPW * NGRAMS
GCHUNK = 128
NCHUNK = IDX_PER_W // GCHUNK
ZROW = IDX_PER_W
NGROUP = BPW // LANES
NDC = EMB_DIM // LANES
TBL_ROWS = 51200

_TAKE_DNUMS = lax.GatherDimensionNumbers(
    offset_dims=(), collapsed_slice_dims=(0,), start_index_map=(0,)
)


def _take(vec, idx):
    return lax.gather(
        vec,
        idx[:, None],
        _TAKE_DNUMS,
        (1,),
        mode=lax.GatherScatterMode.PROMISE_IN_BOUNDS,
    )


def _sc_body(idx_hbm, table_hbm, out_hbm, idx_v, gidx_v, rows_v, out_v, sem,
             gsem):
    wid = lax.axis_index("s") * 2 + lax.axis_index("c")
    base = wid * BPW

    idx_copies = [
        pltpu.make_async_copy(
            idx_hbm.at[pl.ds(s * BATCH + base, BPW)],
            idx_v.at[pl.ds(s * BPW, BPW)],
            sem,
        )
        for s in range(NGRAMS)
    ]
    with jax.named_scope("stage_idx"):
        for c in idx_copies:
            c.start()
        for c in idx_copies:
            c.wait()

    lanes = lax.iota(jnp.int32, LANES)
    with jax.named_scope("gidx"):
        for s in range(NGRAMS):
            for g in range(NGROUP):
                x = idx_v[pl.ds(s * BPW + g * LANES, LANES)]
                plsc.store_scatter(
                    gidx_v,
                    [lanes * NGRAMS + (g * LANES * NGRAMS + s)],
                    jnp.where(x >= TBL_ROWS, x - TBL_ROWS, x),
                )

    row_copies = [
        pltpu.make_async_copy(
            table_hbm.at[gidx_v.at[pl.ds(j * GCHUNK, GCHUNK)]],
            rows_v.at[pl.ds(j * GCHUNK, GCHUNK)],
            gsem,
        )
        for j in range(NCHUNK)
    ]
    for c in row_copies:
        c.start()

    zeros16 = jnp.zeros((LANES,), jnp.float32)
    for dc in range(2 * EMB_DIM // LANES):
        rows_v[ZROW, pl.ds(dc * LANES, LANES)] = zeros16

    with jax.named_scope("masks"):
        groups = []
        for g in range(NGROUP):
            v = [
                idx_v[pl.ds(s * BPW + g * LANES, LANES)] for s in range(NGRAMS)
            ]
            pf = []
            cb = []
            for s in range(NGRAMS):
                dup = None
                for t in range(s):
                    e = v[s] == v[t]
                    dup = e if dup is None else (dup | e)
                row = lanes * NGRAMS + (g * LANES * NGRAMS + s)
                if dup is not None:
                    row = jnp.where(dup, ZROW, row)
                pf.append(row)
                cb.append(jnp.where(v[s] >= TBL_ROWS, EMB_DIM, 0))
            groups.append((pf, cb))

    coffs = [dc * LANES + lanes for dc in range(NDC)]
    waited = 0
    sub = LANES // 2
    for half in range(BPW // sub):
        g = (half * sub) // LANES
        need = min(NCHUNK, -(-((half + 1) * sub * NGRAMS) // GCHUNK))
        with jax.named_scope("gather_wait"):
            while waited < need:
                row_copies[waited].wait()
                waited += 1
        pf, cb = groups[g]
        with jax.named_scope("accum"):

            def bbody(b, _, pf=pf, cb=cb, g=g):
                bsp = jnp.full((LANES,), b, jnp.int32)
                accs = None
                for s in range(NGRAMS):
                    rsp = _take(pf[s], bsp)
                    csp = _take(cb[s], bsp)
                    vals = [
                        plsc.load_gather(rows_v, [rsp, csp + coff])
                        for coff in coffs
                    ]
                    if accs is None:
                        accs = vals
                    else:
                        accs = [a + x for a, x in zip(accs, vals)]
                for dc in range(NDC):
                    out_v[b + g * LANES, pl.ds(dc * LANES, LANES)] = accs[dc]
                return _

            lo = (half * sub) % LANES
            lax.fori_loop(lo, lo + sub, bbody, None)

    with jax.named_scope("writeout"):
        pltpu.sync_copy(out_v, out_hbm.at[pl.ds(base, BPW)])


def _pack_body(a_ref, b_ref, out_ref):
    ab = jnp.concatenate([a_ref[...], b_ref[...]], axis=0)
    eye = jnp.eye(2 * EMB_DIM, dtype=jnp.float32)
    dn = (((0,), (0,)), ((), ()))
    out_ref[...] = lax.dot_general(
        ab, eye, dn, preferred_element_type=jnp.float32
    )


_PACK_CB = 10240


def _pack(Wt):
    nblk = TBL_ROWS // _PACK_CB
    return pl.pallas_call(
        _pack_body,
        grid=(nblk,),
        in_specs=[
            pl.BlockSpec((EMB_DIM, _PACK_CB), lambda i: (0, i)),
            pl.BlockSpec((EMB_DIM, _PACK_CB), lambda i, n=nblk: (0, i + n)),
        ],
        out_specs=pl.BlockSpec((_PACK_CB, 2 * EMB_DIM), lambda i: (i, 0)),
        out_shape=jax.ShapeDtypeStruct((TBL_ROWS, 2 * EMB_DIM), jnp.float32),
        compiler_params=pltpu.CompilerParams(
            dimension_semantics=("parallel",)
        ),
    )(Wt, Wt)


def kernel(input, W):
    idx_lin = jnp.maximum(input.reshape(-1), 0)
    table2 = _pack(W.T)
    mesh = plsc.VectorSubcoreMesh(core_axis_name="c", subcore_axis_name="s")
    f = pl.kernel(
        _sc_body,
        out_type=jax.ShapeDtypeStruct((BATCH, EMB_DIM), jnp.float32),
        mesh=mesh,
        compiler_params=pltpu.CompilerParams(
            needs_layout_passes=False, use_tc_tiling_on_sc=False
        ),
        scratch_types=[
            pltpu.VMEM((IDX_PER_W,), jnp.int32),
            pltpu.VMEM((IDX_PER_W,), jnp.int32),
            pltpu.VMEM((IDX_PER_W + 1, 2 * EMB_DIM), jnp.float32),
            pltpu.VMEM((BPW, EMB_DIM), jnp.float32),
            pltpu.SemaphoreType.DMA,
            pltpu.SemaphoreType.DMA,
        ],
    )
    return f(idx_lin, table2)


# speedup vs baseline: 1.0159x; 1.0022x over previous
import jax
import jax.numpy as jnp
from jax import lax
from jax.experimental import pallas as pl
from jax.experimental.pallas import tpu as pltpu
from jax.experimental.pallas import tpu_sc as plsc

NGRAMS = 20
BATCH = 1024
EMB_DIM = 64
LANES = 16
NW = 32
BPW = BATCH // NW
IDX_PER_W = BPW * NGRAMS
GCHUNK = 128
NCHUNK = IDX_PER_W // GCHUNK
ZROW = IDX_PER_W
NGROUP = BPW // LANES
NDC = EMB_DIM // LANES
TBL_ROWS = 51200

_TAKE_DNUMS = lax.GatherDimensionNumbers(
    offset_dims=(), collapsed_slice_dims=(0,), start_index_map=(0,)
)


def _take(vec, idx):
    return lax.gather(
        vec,
        idx[:, None],
        _TAKE_DNUMS,
        (1,),
        mode=lax.GatherScatterMode.PROMISE_IN_BOUNDS,
    )


def _sc_body(idx_hbm, table_hbm, out_hbm, idx_v, gidx_v, rows_v, out_v, sem,
             gsem):
    wid = lax.axis_index("s") * 2 + lax.axis_index("c")
    base = wid * BPW

    idx_copies = [
        pltpu.make_async_copy(
            idx_hbm.at[s, pl.ds(base, BPW)],
            idx_v.at[pl.ds(s * BPW, BPW)],
            sem,
        )
        for s in range(NGRAMS)
    ]
    with jax.named_scope("stage_idx"):
        for c in idx_copies:
            c.start()
        for c in idx_copies:
            c.wait()

    lanes = lax.iota(jnp.int32, LANES)
    with jax.named_scope("gidx"):
        for s in range(NGRAMS):
            for g in range(NGROUP):
                x = idx_v[pl.ds(s * BPW + g * LANES, LANES)]
                plsc.store_scatter(
                    gidx_v,
                    [lanes * NGRAMS + (g * LANES * NGRAMS + s)],
                    jnp.where(x >= TBL_ROWS, x - TBL_ROWS, x),
                )

    row_copies = [
        pltpu.make_async_copy(
            table_hbm.at[gidx_v.at[pl.ds(j * GCHUNK, GCHUNK)]],
            rows_v.at[pl.ds(j * GCHUNK, GCHUNK)],
            gsem,
        )
        for j in range(NCHUNK)
    ]
    for c in row_copies:
        c.start()

    zeros16 = jnp.zeros((LANES,), jnp.float32)
    for dc in range(2 * EMB_DIM // LANES):
        rows_v[ZROW, pl.ds(dc * LANES, LANES)] = zeros16

    with jax.named_scope("masks"):
        groups = []
        for g in range(NGROUP):
            v = [
                idx_v[pl.ds(s * BPW + g * LANES, LANES)] for s in range(NGRAMS)
            ]
            pf = []
            cb = []
            for s in range(NGRAMS):
                dup = None
                for t in range(s):
                    e = v[s] == v[t]
                    dup = e if dup is None else (dup | e)
                row = lanes * NGRAMS + (g * LANES * NGRAMS + s)
                if dup is not None:
                    row = jnp.where(dup, ZROW, row)
                pf.append(row)
                cb.append(jnp.where(v[s] >= TBL_ROWS, EMB_DIM, 0))
            groups.append((pf, cb))

    coffs = [dc * LANES + lanes for dc in range(NDC)]
    waited = 0
    sub = LANES // 2
    for half in range(BPW // sub):
        g = (half * sub) // LANES
        need = min(NCHUNK, -(-((half + 1) * sub * NGRAMS) // GCHUNK))
        with jax.named_scope("gather_wait"):
            while waited < need:
                row_copies[waited].wait()
                waited += 1
        pf, cb = groups[g]
        with jax.named_scope("accum"):

            def bbody(b, _, pf=pf, cb=cb, g=g):
                bsp = jnp.full((LANES,), b, jnp.int32)
                accs = None
                for s in range(NGRAMS):
                    rsp = _take(pf[s], bsp)
                    csp = _take(cb[s], bsp)
                    vals = [
                        plsc.load_gather(rows_v, [rsp, csp + coff])
                        for coff in coffs
                    ]
                    if accs is None:
                        accs = vals
                    else:
                        accs = [a + x for a, x in zip(accs, vals)]
                for dc in range(NDC):
                    out_v[b + g * LANES, pl.ds(dc * LANES, LANES)] = accs[dc]
                return _

            lo = (half * sub) % LANES
            lax.fori_loop(lo, lo + sub, bbody, None)

    with jax.named_scope("writeout"):
        pltpu.sync_copy(out_v, out_hbm.at[pl.ds(base, BPW)])


def _pack_body(a_ref, b_ref, out_ref):
    ab = jnp.concatenate([a_ref[...], b_ref[...]], axis=0)
    eye = jnp.eye(2 * EMB_DIM, dtype=jnp.float32)
    dn = (((0,), (0,)), ((), ()))
    out_ref[...] = lax.dot_general(
        ab, eye, dn, preferred_element_type=jnp.float32
    )


_PACK_CB = 10240


def _pack(Wt):
    nblk = TBL_ROWS // _PACK_CB
    return pl.pallas_call(
        _pack_body,
        grid=(nblk,),
        in_specs=[
            pl.BlockSpec((EMB_DIM, _PACK_CB), lambda i: (0, i)),
            pl.BlockSpec((EMB_DIM, _PACK_CB), lambda i, n=nblk: (0, i + n)),
        ],
        out_specs=pl.BlockSpec((_PACK_CB, 2 * EMB_DIM), lambda i: (i, 0)),
        out_shape=jax.ShapeDtypeStruct((TBL_ROWS, 2 * EMB_DIM), jnp.float32),
        compiler_params=pltpu.CompilerParams(
            dimension_semantics=("parallel",)
        ),
    )(Wt, Wt)


def kernel(input, W):
    table2 = _pack(W.T)
    mesh = plsc.VectorSubcoreMesh(core_axis_name="c", subcore_axis_name="s")
    f = pl.kernel(
        _sc_body,
        out_type=jax.ShapeDtypeStruct((BATCH, EMB_DIM), jnp.float32),
        mesh=mesh,
        compiler_params=pltpu.CompilerParams(
            needs_layout_passes=False, use_tc_tiling_on_sc=False
        ),
        scratch_types=[
            pltpu.VMEM((IDX_PER_W,), jnp.int32),
            pltpu.VMEM((IDX_PER_W,), jnp.int32),
            pltpu.VMEM((IDX_PER_W + 1, 2 * EMB_DIM), jnp.float32),
            pltpu.VMEM((BPW, EMB_DIM), jnp.float32),
            pltpu.SemaphoreType.DMA,
            pltpu.SemaphoreType.DMA,
        ],
    )
    return f(input, table2)
